# Initial kernel scaffold; baseline (speedup 1.0000x reference)
import functools

import jax
import jax.numpy as jnp
from jax import lax
from jax.experimental import pallas as pl
from jax.experimental.pallas import tpu as pltpu
from jax.experimental.pallas import tpu_sc as plsc

F = 26
V = 100000
E = 16
B = 16384
BF = B * F

NC = 2
NS = 16
NW = NC * NS
PER_W = BF // NW
G = 128
NG = PER_W // G
GRP = 8
NGRP = NG // GRP


def _sc_gather(flat_tables, ids_2d):
    mesh = plsc.VectorSubcoreMesh(core_axis_name="c", subcore_axis_name="s")

    @functools.partial(
        pl.kernel,
        out_type=jax.ShapeDtypeStruct((BF, E), jnp.float32),
        mesh=mesh,
        scratch_types=[
            pltpu.VMEM((NG, G), jnp.int32),
            pltpu.VMEM((GRP * G, E), jnp.float32),
            pltpu.SemaphoreType.DMA,
            pltpu.SemaphoreType.DMA,
        ],
    )
    def k(tab_hbm, ids_hbm, out_hbm, idx_v, buf, gsem, osem):
        wid = lax.axis_index("s") * NC + lax.axis_index("c")
        base = wid * PER_W
        pltpu.sync_copy(ids_hbm.at[pl.ds(wid * NG, NG)], idx_v)

        iota = lax.iota(jnp.int32, 16)

        def add_body(g, carry):
            for l in range(G // 16):
                sl = pl.ds(l * 16, 16)
                pos = jnp.broadcast_to(g * G + l * 16, (16,)) + iota
                off = (pos % F) * V
                idx_v[g, sl] = idx_v[g, sl] + off
            return carry

        lax.fori_loop(0, NG, add_body, 0)

        def grp_body(gp, carry):
            handles = []
            for j in range(GRP):
                h = pltpu.async_copy(
                    tab_hbm.at[idx_v.at[gp * GRP + j]],
                    buf.at[pl.ds(j * G, G)],
                    gsem,
                )
                handles.append(h)
            for h in handles:
                h.wait()
            out = pltpu.async_copy(
                buf, out_hbm.at[pl.ds(base + gp * (GRP * G), GRP * G)], osem
            )
            out.wait()
            return carry

        lax.fori_loop(0, NGRP, grp_body, 0)

    return k(flat_tables, ids_2d)


def _tc_mlp(x, W1, b1, W2, b2, W3, b3):
    BLK = 1024
    grid = B // BLK

    def body(x_ref, w1_ref, b1_ref, w2_ref, b2_ref, w3_ref, b3_ref, o_ref):
        xb = x_ref[...]
        h = jnp.dot(xb, w1_ref[...], preferred_element_type=jnp.float32)
        h = jnp.maximum(h + b1_ref[...], 0.0)
        h = jnp.dot(h, w2_ref[...], preferred_element_type=jnp.float32)
        h = jnp.maximum(h + b2_ref[...], 0.0)
        o = jnp.dot(h, w3_ref[...], preferred_element_type=jnp.float32)
        o_ref[...] = jax.nn.sigmoid(o + b3_ref[...])

    out = pl.pallas_call(
        body,
        grid=(grid,),
        in_specs=[
            pl.BlockSpec((BLK, F * E), lambda i: (i, 0)),
            pl.BlockSpec((F * E, 256), lambda i: (0, 0)),
            pl.BlockSpec((1, 256), lambda i: (0, 0)),
            pl.BlockSpec((256, 128), lambda i: (0, 0)),
            pl.BlockSpec((1, 128), lambda i: (0, 0)),
            pl.BlockSpec((128, 1), lambda i: (0, 0)),
            pl.BlockSpec((1, 1), lambda i: (0, 0)),
        ],
        out_specs=pl.BlockSpec((BLK, 1), lambda i: (i, 0)),
        out_shape=jax.ShapeDtypeStruct((B, 1), jnp.float32),
    )(x, W1, b1.reshape(1, 256), W2, b2.reshape(1, 128), W3, b3.reshape(1, 1))
    return out[:, 0]


def kernel(ids, tables, W1, b1, W2, b2, W3, b3):
    ids_2d = ids.astype(jnp.int32).reshape(BF // G, G)
    flat_tables = tables.reshape(F * V, E)
    rows = _sc_gather(flat_tables, ids_2d)
    x = rows.reshape(B, F * E)
    return _tc_mlp(x, W1, b1, W2, b2, W3, b3)


# speedup vs baseline: 7.7853x; 7.7853x over previous
import functools

import jax
import jax.numpy as jnp
from jax import lax
from jax.experimental import pallas as pl
from jax.experimental.pallas import tpu as pltpu
from jax.experimental.pallas import tpu_sc as plsc

F = 26
V = 100000
E = 16
B = 16384
BF = B * F

NC = 2
NS = 16
NW = NC * NS
PER_W = BF // NW
G = 128
NG = PER_W // G
GRP = 8
NGRP = NG // GRP


def _sc_gather(flat_tables, ids_2d):
    mesh = plsc.VectorSubcoreMesh(core_axis_name="c", subcore_axis_name="s")

    @functools.partial(
        pl.kernel,
        out_type=jax.ShapeDtypeStruct((BF, E), jnp.float32),
        mesh=mesh,
        scratch_types=[
            pltpu.VMEM((NG, G), jnp.int32),
            pltpu.VMEM((GRP * G, E), jnp.float32),
            pltpu.SemaphoreType.DMA,
            pltpu.SemaphoreType.DMA,
        ],
        compiler_params=pltpu.CompilerParams(use_tc_tiling_on_sc=False),
    )
    def k(tab_hbm, ids_hbm, out_hbm, idx_v, buf, gsem, osem):
        wid = lax.axis_index("s") * NC + lax.axis_index("c")
        base = wid * PER_W
        pltpu.sync_copy(ids_hbm.at[pl.ds(wid * NG, NG)], idx_v)

        iota = lax.iota(jnp.int32, 16)

        def add_body(g, carry):
            for l in range(G // 16):
                sl = pl.ds(l * 16, 16)
                pos = jnp.broadcast_to(g * G + l * 16, (16,)) + iota
                off = (pos % F) * V
                idx_v[g, sl] = idx_v[g, sl] + off
            return carry

        lax.fori_loop(0, NG, add_body, 0)

        def grp_body(gp, carry):
            handles = []
            for j in range(GRP):
                h = pltpu.async_copy(
                    tab_hbm.at[idx_v.at[gp * GRP + j]],
                    buf.at[pl.ds(j * G, G)],
                    gsem,
                )
                handles.append(h)
            for h in handles:
                h.wait()
            out = pltpu.async_copy(
                buf, out_hbm.at[pl.ds(base + gp * (GRP * G), GRP * G)], osem
            )
            out.wait()
            return carry

        lax.fori_loop(0, NGRP, grp_body, 0)

    return k(flat_tables, ids_2d)


def _tc_mlp(x, W1, b1, W2, b2, W3, b3):
    BLK = 1024
    grid = B // BLK

    def body(x_ref, w1_ref, b1_ref, w2_ref, b2_ref, w3_ref, b3_ref, o_ref):
        xb = x_ref[...]
        h = jnp.dot(xb, w1_ref[...], preferred_element_type=jnp.float32)
        h = jnp.maximum(h + b1_ref[...], 0.0)
        h = jnp.dot(h, w2_ref[...], preferred_element_type=jnp.float32)
        h = jnp.maximum(h + b2_ref[...], 0.0)
        o = jnp.dot(h, w3_ref[...], preferred_element_type=jnp.float32)
        o_ref[...] = jax.nn.sigmoid(o + b3_ref[...])

    out = pl.pallas_call(
        body,
        grid=(grid,),
        in_specs=[
            pl.BlockSpec((BLK, F * E), lambda i: (i, 0)),
            pl.BlockSpec((F * E, 256), lambda i: (0, 0)),
            pl.BlockSpec((1, 256), lambda i: (0, 0)),
            pl.BlockSpec((256, 128), lambda i: (0, 0)),
            pl.BlockSpec((1, 128), lambda i: (0, 0)),
            pl.BlockSpec((128, 1), lambda i: (0, 0)),
            pl.BlockSpec((1, 1), lambda i: (0, 0)),
        ],
        out_specs=pl.BlockSpec((BLK, 1), lambda i: (i, 0)),
        out_shape=jax.ShapeDtypeStruct((B, 1), jnp.float32),
    )(x, W1, b1.reshape(1, 256), W2, b2.reshape(1, 128), W3, b3.reshape(1, 1))
    return out[:, 0]


def kernel(ids, tables, W1, b1, W2, b2, W3, b3):
    ids_2d = ids.astype(jnp.int32).reshape(BF // G, G)
    flat_tables = tables.reshape(F * V, E)
    rows = _sc_gather(flat_tables, ids_2d)
    x = rows.reshape(B, F * E)
    return _tc_mlp(x, W1, b1, W2, b2, W3, b3)
